# baseline (speedup 1.0000x reference)
import jax
import jax.numpy as jnp
from jax.experimental import pallas as pl
from jax.experimental.pallas import tpu as pltpu

_LANES = 128


def _gather_body(x_ref, i_ref, o_ref):
    T = x_ref.shape[0]
    sub = 256 if T % 256 == 0 else T

    for rb in range(T // sub):
        rows = slice(rb * sub, (rb + 1) * sub)
        x = x_ref[rows, :]
        idx = i_ref[rows, :]
        lo = jnp.bitwise_and(idx, _LANES - 1)
        g0 = jnp.take_along_axis(x[:, 0:128], lo, axis=1)
        g1 = jnp.take_along_axis(x[:, 128:256], lo, axis=1)
        g2 = jnp.take_along_axis(x[:, 256:384], lo, axis=1)
        g3 = jnp.take_along_axis(x[:, 384:512], lo, axis=1)
        t01 = jnp.where(idx < 128, g0, g1)
        t23 = jnp.where(idx < 384, g2, g3)
        o_ref[rows, :] = jnp.where(idx < 256, t01, t23)


def _gather_2d(x2d, idx2d, tile_r=1024):
    R, M = x2d.shape
    _, P = idx2d.shape
    assert M % _LANES == 0 and P % _LANES == 0 and R % tile_r == 0

    grid = (R // tile_r, P // _LANES)
    return pl.pallas_call(
        _gather_body,
        out_shape=jax.ShapeDtypeStruct((R, P), x2d.dtype),
        grid=grid,
        in_specs=[
            pl.BlockSpec((tile_r, M), lambda i, j: (i, 0)),
            pl.BlockSpec((tile_r, _LANES), lambda i, j: (i, j)),
        ],
        out_specs=pl.BlockSpec((tile_r, _LANES), lambda i, j: (i, j)),
        compiler_params=pltpu.CompilerParams(
            dimension_semantics=("arbitrary", "arbitrary"),
            vmem_limit_bytes=60 * 1024 * 1024,
        ),
    )(x2d, idx2d)


def kernel(inp, index):
    batch_shape = index.shape[:-1]
    M = inp.shape[-1]
    P = index.shape[-1]
    x2 = inp.reshape(-1, M)
    i2 = index.reshape(-1, P).astype(jnp.int32)
    out2 = _gather_2d(x2, i2)
    return out2.reshape(*batch_shape, P).astype(inp.dtype)


# speedup vs baseline: 1.1813x; 1.1813x over previous
import jax
import jax.numpy as jnp
from jax.experimental import pallas as pl
from jax.experimental.pallas import tpu as pltpu

_LANES = 128


def _gather_body(x_ref, i_ref, o_ref):
    T = x_ref.shape[0]
    n_p = i_ref.shape[1] // _LANES
    sub = 256 if T % 256 == 0 else T

    for rb in range(T // sub):
        rows = slice(rb * sub, (rb + 1) * sub)
        x = x_ref[rows, :]
        idx = i_ref[rows, :]
        lo = jnp.bitwise_and(idx, _LANES - 1)
        for h in range(n_p):
            sl = slice(h * _LANES, (h + 1) * _LANES)
            lo_h = lo[:, sl]
            idx_h = idx[:, sl]
            g0 = jnp.take_along_axis(x[:, 0:128], lo_h, axis=1)
            g1 = jnp.take_along_axis(x[:, 128:256], lo_h, axis=1)
            g2 = jnp.take_along_axis(x[:, 256:384], lo_h, axis=1)
            g3 = jnp.take_along_axis(x[:, 384:512], lo_h, axis=1)
            t01 = jnp.where(idx_h < 128, g0, g1)
            t23 = jnp.where(idx_h < 384, g2, g3)
            o_ref[rows, sl] = jnp.where(idx_h < 256, t01, t23)


def _gather_2d(x2d, idx2d, tile_r=1024):
    R, M = x2d.shape
    _, P = idx2d.shape
    assert M % _LANES == 0 and P % _LANES == 0 and R % tile_r == 0

    grid = (R // tile_r,)
    return pl.pallas_call(
        _gather_body,
        out_shape=jax.ShapeDtypeStruct((R, P), x2d.dtype),
        grid=grid,
        in_specs=[
            pl.BlockSpec((tile_r, M), lambda i: (i, 0)),
            pl.BlockSpec((tile_r, P), lambda i: (i, 0)),
        ],
        out_specs=pl.BlockSpec((tile_r, P), lambda i: (i, 0)),
        compiler_params=pltpu.CompilerParams(
            dimension_semantics=("arbitrary",),
        ),
    )(x2d, idx2d)


def kernel(inp, index):
    batch_shape = index.shape[:-1]
    M = inp.shape[-1]
    P = index.shape[-1]
    x2 = inp.reshape(-1, M)
    i2 = index.reshape(-1, P).astype(jnp.int32)
    out2 = _gather_2d(x2, i2)
    return out2.reshape(*batch_shape, P).astype(inp.dtype)
